# baseline (device time: 161720 ns/iter reference)
import jax
import jax.numpy as jnp
from jax import lax
from jax.experimental import pallas as pl
from jax.experimental.pallas import tpu as pltpu

N_DEV = 4


def kernel(x, w_mat, scale_x, scale_w):
    m_tot, k_loc = x.shape
    k_tot, n_tot = w_mat.shape
    m_loc = m_tot // N_DEV
    assert m_loc == k_loc

    BN = 1024
    n_blocks = n_tot // BN

    def body(x_ref, w_ref, sx_ref, sw_ref, out_ref,
             x8_ref, gat_ref, gat16_ref, send_sems, recv_sems):
        n = pl.program_id(0)
        j = pl.program_id(1)
        my_i = lax.axis_index("i")

        @pl.when(jnp.logical_and(n == 0, j == 0))
        def _init():
            x8_ref[:, :] = x_ref[:, :].astype(jnp.float8_e4m3fn)
            gat16_ref[my_i] = x_ref[pl.ds(my_i * m_loc, m_loc), :].astype(
                jnp.bfloat16)
            rdmas = []
            for d in range(1, N_DEV):
                peer = lax.rem(my_i + d, N_DEV)
                rdma = pltpu.make_async_remote_copy(
                    src_ref=x8_ref.at[pl.ds(peer * m_loc, m_loc), :],
                    dst_ref=gat_ref.at[my_i],
                    send_sem=send_sems.at[peer],
                    recv_sem=recv_sems.at[my_i],
                    device_id=(peer,),
                    device_id_type=pl.DeviceIdType.MESH,
                )
                rdma.start()
                rdmas.append(rdma)
            for rdma in rdmas:
                rdma.wait_send()

        @pl.when(jnp.logical_and(n == 0, j != my_i))
        def _wait():
            recv = pltpu.make_async_remote_copy(
                src_ref=gat_ref.at[j],
                dst_ref=gat_ref.at[j],
                send_sem=send_sems.at[j],
                recv_sem=recv_sems.at[j],
                device_id=(j,),
                device_id_type=pl.DeviceIdType.MESH,
            )
            recv.wait_recv()
            gat16_ref[j] = gat_ref[j].astype(jnp.bfloat16)

        w16 = w_ref[:, :].astype(jnp.bfloat16)
        acc = lax.dot_general(
            gat16_ref[j], w16,
            dimension_numbers=(((1,), (0,)), ((), ())),
            preferred_element_type=jnp.float32,
        )

        @pl.when(j == 0)
        def _store():
            out_ref[:, :] = acc

        @pl.when(j != 0)
        def _accum():
            out_ref[:, :] = out_ref[:, :] + acc

        @pl.when(j == N_DEV - 1)
        def _epilogue():
            s = sx_ref[0] * sw_ref[0]
            out_ref[:, :] = jnp.maximum(out_ref[:, :] * s, 0.0)

    return pl.pallas_call(
        body,
        grid=(n_blocks, N_DEV),
        in_specs=[
            pl.BlockSpec(memory_space=pltpu.VMEM),
            pl.BlockSpec((k_loc, BN), lambda n, j: (j, n)),
            pl.BlockSpec(memory_space=pltpu.SMEM),
            pl.BlockSpec(memory_space=pltpu.SMEM),
        ],
        out_specs=pl.BlockSpec((m_loc, BN), lambda n, j: (0, n)),
        out_shape=jax.ShapeDtypeStruct((m_loc, n_tot), jnp.float32),
        scratch_shapes=[
            pltpu.VMEM((m_tot, k_loc), jnp.float8_e4m3fn),
            pltpu.VMEM((N_DEV, m_loc, k_loc), jnp.float8_e4m3fn),
            pltpu.VMEM((N_DEV, m_loc, k_loc), jnp.bfloat16),
            pltpu.SemaphoreType.DMA((N_DEV,)),
            pltpu.SemaphoreType.DMA((N_DEV,)),
        ],
        compiler_params=pltpu.CompilerParams(
            dimension_semantics=("arbitrary", "arbitrary"),
            vmem_limit_bytes=96 * 1024 * 1024,
        ),
    )(x, w_mat, scale_x, scale_w)


# device time: 135785 ns/iter; 1.1910x vs baseline; 1.1910x over previous
import jax
import jax.numpy as jnp
from jax import lax
from jax.experimental import pallas as pl
from jax.experimental.pallas import tpu as pltpu

N_DEV = 4
N_HALF = 2


def kernel(x, w_mat, scale_x, scale_w):
    m_tot, k_loc = x.shape
    k_tot, n_tot = w_mat.shape
    m_loc = m_tot // N_DEV
    assert m_loc == k_loc

    BN = 1024
    n_blocks = n_tot // BN
    nb_half = n_blocks // N_HALF
    n_steps = N_DEV * n_blocks

    e4m3 = jnp.float8_e4m3fn
    e5m2 = jnp.float8_e5m2

    def body(x_ref, w_ref, sx_ref, sw_ref, out_ref,
             x8_ref, gat_ref, out_v, wraw, wcv,
             send_sems, recv_sems, wsems, osems):
        h = pl.program_id(0)
        g = pl.program_id(1)
        nb = pl.program_id(2)
        t = (h * N_DEV + g) * nb_half + nb
        my_i = lax.axis_index("i")

        def w_tile_start(step, slot):
            g_ = (step // nb_half) % N_DEV
            ncol_ = (step // (N_DEV * nb_half)) * nb_half + step % nb_half
            kb_ = lax.rem(my_i + g_, N_DEV)
            pltpu.make_async_copy(
                w_ref.at[pl.ds(kb_ * k_loc, k_loc), pl.ds(ncol_ * BN, BN)],
                wraw.at[slot],
                wsems.at[slot],
            ).start()

        def w_tile_wait(slot):
            pltpu.make_async_copy(
                w_ref.at[pl.ds(0, k_loc), pl.ds(0, BN)],
                wraw.at[slot],
                wsems.at[slot],
            ).wait()

        @pl.when(t == 0)
        def _prologue():
            x8_ref[:, :] = x_ref[:, :].astype(e4m3)
            gat_ref[my_i] = x8_ref[pl.ds(my_i * m_loc, m_loc), :]
            for d in range(1, N_DEV):
                peer = lax.rem(my_i + d, N_DEV)
                pltpu.make_async_remote_copy(
                    src_ref=x8_ref.at[pl.ds(peer * m_loc, m_loc), :],
                    dst_ref=gat_ref.at[my_i],
                    send_sem=send_sems.at[peer],
                    recv_sem=recv_sems.at[my_i],
                    device_id=(peer,),
                    device_id_type=pl.DeviceIdType.MESH,
                ).start()
            w_tile_start(0, 0)
            w_tile_wait(0)
            wcv[0] = wraw[0].astype(e5m2)
            w_tile_start(1, 1)

        kb = lax.rem(my_i + g, N_DEV)
        slot = lax.rem(nb, 2)
        nslot = lax.rem(nb + 1, 2)

        @pl.when(jnp.logical_and(h == 0, jnp.logical_and(g > 0, nb == 0)))
        def _wait_recv():
            pltpu.make_async_remote_copy(
                src_ref=gat_ref.at[kb],
                dst_ref=gat_ref.at[kb],
                send_sem=send_sems.at[kb],
                recv_sem=recv_sems.at[kb],
                device_id=(kb,),
                device_id_type=pl.DeviceIdType.MESH,
            ).wait_recv()

        @pl.when(jnp.logical_and(h == 1, g == 0))
        def _reuse_wait():
            pltpu.make_async_copy(
                out_v.at[:, pl.ds(nb * BN, BN)],
                out_ref.at[:, pl.ds(nb * BN, BN)],
                osems.at[nb],
            ).wait()

        acc = lax.dot_general(
            gat_ref[kb], wcv[slot],
            dimension_numbers=(((1,), (0,)), ((), ())),
            preferred_element_type=jnp.float32,
        )
        osl = pl.ds(nb * BN, BN)

        @pl.when(g == 0)
        def _store():
            out_v[:, osl] = acc

        @pl.when(g != 0)
        def _accum():
            out_v[:, osl] = out_v[:, osl] + acc

        @pl.when(t < n_steps - 1)
        def _advance():
            w_tile_wait(nslot)
            wcv[nslot] = wraw[nslot].astype(e5m2)

            @pl.when(t < n_steps - 2)
            def _prefetch():
                w_tile_start(t + 2, slot)

        @pl.when(g == N_DEV - 1)
        def _epilogue():
            s = sx_ref[0] * sw_ref[0]
            out_v[:, osl] = jnp.maximum(out_v[:, osl] * s, 0.0)
            ncol = h * nb_half + nb
            pltpu.make_async_copy(
                out_v.at[:, osl],
                out_ref.at[:, pl.ds(ncol * BN, BN)],
                osems.at[nb],
            ).start()

        @pl.when(t == n_steps - 1)
        def _drain():
            for b in range(nb_half):
                sl = pl.ds(b * BN, BN)
                pltpu.make_async_copy(
                    out_v.at[:, sl], out_ref.at[:, sl], osems.at[b],
                ).wait()
            for d in range(1, N_DEV):
                peer = lax.rem(my_i + d, N_DEV)
                pltpu.make_async_remote_copy(
                    src_ref=x8_ref.at[pl.ds(peer * m_loc, m_loc), :],
                    dst_ref=gat_ref.at[my_i],
                    send_sem=send_sems.at[peer],
                    recv_sem=recv_sems.at[my_i],
                    device_id=(peer,),
                    device_id_type=pl.DeviceIdType.MESH,
                ).wait_send()

    return pl.pallas_call(
        body,
        grid=(N_HALF, N_DEV, nb_half),
        in_specs=[
            pl.BlockSpec(memory_space=pltpu.VMEM),
            pl.BlockSpec(memory_space=pl.ANY),
            pl.BlockSpec(memory_space=pltpu.SMEM),
            pl.BlockSpec(memory_space=pltpu.SMEM),
        ],
        out_specs=pl.BlockSpec(memory_space=pl.ANY),
        out_shape=jax.ShapeDtypeStruct((m_loc, n_tot), jnp.float32),
        scratch_shapes=[
            pltpu.VMEM((m_tot, k_loc), e4m3),
            pltpu.VMEM((N_DEV, m_loc, k_loc), e4m3),
            pltpu.VMEM((m_loc, n_tot // N_HALF), jnp.float32),
            pltpu.VMEM((2, k_loc, BN), jnp.float32),
            pltpu.VMEM((2, k_loc, BN), e5m2),
            pltpu.SemaphoreType.DMA((N_DEV,)),
            pltpu.SemaphoreType.DMA((N_DEV,)),
            pltpu.SemaphoreType.DMA((2,)),
            pltpu.SemaphoreType.DMA((nb_half,)),
        ],
        compiler_params=pltpu.CompilerParams(
            dimension_semantics=("arbitrary", "arbitrary", "arbitrary"),
            vmem_limit_bytes=64 * 1024 * 1024,
        ),
    )(x, w_mat, scale_x, scale_w)


# device time: 124401 ns/iter; 1.3000x vs baseline; 1.0915x over previous
import jax
import jax.numpy as jnp
from jax import lax
from jax.experimental import pallas as pl
from jax.experimental.pallas import tpu as pltpu

N_DEV = 4
N_HALF = 2
KH = 2


def kernel(x, w_mat, scale_x, scale_w):
    m_tot, k_loc = x.shape
    k_tot, n_tot = w_mat.shape
    m_loc = m_tot // N_DEV
    assert m_loc == k_loc

    BN = 1024
    KK = KH * k_loc
    n_blocks = n_tot // BN
    nb_half = n_blocks // N_HALF
    n_kp = N_DEV // KH
    n_steps = N_HALF * n_kp * nb_half

    e4m3 = jnp.float8_e4m3fn
    e5m2 = jnp.float8_e5m2

    def body(x_ref, w_ref, sx_ref, sw_ref, out_ref,
             x8_ref, xg_ref, out_v, xstg, wraw, wcv,
             send_sems, recv_sems, xsems, wsems, osems):
        h = pl.program_id(0)
        kp = pl.program_id(1)
        nb = pl.program_id(2)
        t = (h * n_kp + kp) * nb_half + nb
        my_i = lax.axis_index("i")

        def w_tile_start(step, slot):
            h_ = step // (n_kp * nb_half)
            kp_ = (step // nb_half) % n_kp
            nb_ = step % nb_half
            ncol_ = h_ * nb_half + nb_
            for half in range(KH):
                kb_ = lax.rem(my_i + kp_ * KH + half, N_DEV)
                pltpu.make_async_copy(
                    w_ref.at[pl.ds(kb_ * k_loc, k_loc),
                             pl.ds(ncol_ * BN, BN)],
                    wraw.at[slot, pl.ds(half * k_loc, k_loc), :],
                    wsems.at[slot, half],
                ).start()

        def w_tile_wait(slot):
            for half in range(KH):
                pltpu.make_async_copy(
                    w_ref.at[pl.ds(0, k_loc), pl.ds(0, BN)],
                    wraw.at[slot, pl.ds(half * k_loc, k_loc), :],
                    wsems.at[slot, half],
                ).wait()

        def recv_desc(sl):
            return pltpu.make_async_remote_copy(
                src_ref=xg_ref.at[:, pl.ds(sl * k_loc, k_loc)],
                dst_ref=xg_ref.at[:, pl.ds(sl * k_loc, k_loc)],
                send_sem=send_sems.at[0],
                recv_sem=recv_sems.at[sl],
                device_id=(0,),
                device_id_type=pl.DeviceIdType.MESH,
            )

        @pl.when(t == 0)
        def _prologue():
            for r in range(2):
                pltpu.make_async_copy(
                    x_ref.at[pl.ds(r * m_loc, m_loc), :],
                    xstg.at[r], xsems.at[r],
                ).start()
            for r in range(N_DEV):
                sl = r % 2
                pltpu.make_async_copy(
                    x_ref.at[pl.ds(r * m_loc, m_loc), :],
                    xstg.at[sl], xsems.at[sl],
                ).wait()
                x8_ref[pl.ds(r * m_loc, m_loc), :] = xstg[sl].astype(e4m3)
                if r + 2 < N_DEV:
                    pltpu.make_async_copy(
                        x_ref.at[pl.ds((r + 2) * m_loc, m_loc), :],
                        xstg.at[sl], xsems.at[sl],
                    ).start()
            xg_ref[:, pl.ds(0, k_loc)] = x8_ref[pl.ds(my_i * m_loc, m_loc), :]
            for d in (3, 2, 1):
                peer = lax.rem(my_i + d, N_DEV)
                pltpu.make_async_remote_copy(
                    src_ref=x8_ref.at[pl.ds(peer * m_loc, m_loc), :],
                    dst_ref=xg_ref.at[:, pl.ds(((N_DEV - d) % N_DEV) * k_loc,
                                               k_loc)],
                    send_sem=send_sems.at[d - 1],
                    recv_sem=recv_sems.at[(N_DEV - d) % N_DEV],
                    device_id=(peer,),
                    device_id_type=pl.DeviceIdType.MESH,
                ).start()
            w_tile_start(0, 0)
            w_tile_wait(0)
            wcv[0] = wraw[0].astype(e5m2)
            w_tile_start(1, 1)

        slot = lax.rem(t, 2)
        nslot = lax.rem(t + 1, 2)

        @pl.when(t == 0)
        def _wait_recv_kp0():
            recv_desc(1).wait_recv()

        @pl.when(t == nb_half)
        def _wait_recv_kp1():
            recv_desc(2).wait_recv()
            recv_desc(3).wait_recv()

        @pl.when(jnp.logical_and(h == 1, kp == 0))
        def _reuse_wait():
            pltpu.make_async_copy(
                out_v.at[:, pl.ds(nb * BN, BN)],
                out_ref.at[:, pl.ds(nb * BN, BN)],
                osems.at[nb],
            ).wait()

        acc = lax.dot_general(
            xg_ref[:, pl.ds(kp * KK, KK)], wcv[slot],
            dimension_numbers=(((1,), (0,)), ((), ())),
            preferred_element_type=jnp.float32,
        )
        osl = pl.ds(nb * BN, BN)

        @pl.when(kp == 0)
        def _store():
            out_v[:, osl] = acc

        @pl.when(kp != 0)
        def _accum():
            out_v[:, osl] = out_v[:, osl] + acc

        @pl.when(t < n_steps - 1)
        def _advance():
            w_tile_wait(nslot)
            wcv[nslot] = wraw[nslot].astype(e5m2)

            @pl.when(t < n_steps - 2)
            def _prefetch():
                w_tile_start(t + 2, slot)

        @pl.when(kp == n_kp - 1)
        def _epilogue():
            s = sx_ref[0] * sw_ref[0]
            out_v[:, osl] = jnp.maximum(out_v[:, osl] * s, 0.0)

        half_steps = n_steps // N_HALF
        for h_ in range(N_HALF):
            @pl.when(t == (h_ + 1) * half_steps - 1)
            def _flush(h_=h_):
                for b in range(nb_half):
                    pltpu.make_async_copy(
                        out_v.at[:, pl.ds(b * BN, BN)],
                        out_ref.at[:, pl.ds((h_ * nb_half + b) * BN, BN)],
                        osems.at[b],
                    ).start()

        @pl.when(t == n_steps - 1)
        def _drain():
            for b in range(nb_half):
                pltpu.make_async_copy(
                    out_v.at[:, pl.ds(b * BN, BN)],
                    out_ref.at[:, pl.ds(b * BN, BN)],
                    osems.at[b],
                ).wait()
            for d in (3, 2, 1):
                peer = lax.rem(my_i + d, N_DEV)
                pltpu.make_async_remote_copy(
                    src_ref=x8_ref.at[pl.ds(peer * m_loc, m_loc), :],
                    dst_ref=xg_ref.at[:, pl.ds(((N_DEV - d) % N_DEV) * k_loc,
                                               k_loc)],
                    send_sem=send_sems.at[d - 1],
                    recv_sem=recv_sems.at[(N_DEV - d) % N_DEV],
                    device_id=(peer,),
                    device_id_type=pl.DeviceIdType.MESH,
                ).wait_send()

    return pl.pallas_call(
        body,
        grid=(N_HALF, n_kp, nb_half),
        in_specs=[
            pl.BlockSpec(memory_space=pl.ANY),
            pl.BlockSpec(memory_space=pl.ANY),
            pl.BlockSpec(memory_space=pltpu.SMEM),
            pl.BlockSpec(memory_space=pltpu.SMEM),
        ],
        out_specs=pl.BlockSpec(memory_space=pl.ANY),
        out_shape=jax.ShapeDtypeStruct((m_loc, n_tot), jnp.float32),
        scratch_shapes=[
            pltpu.VMEM((m_tot, k_loc), e4m3),
            pltpu.VMEM((m_loc, m_tot), e4m3),
            pltpu.VMEM((m_loc, n_tot // N_HALF), jnp.float32),
            pltpu.VMEM((2, m_loc, k_loc), jnp.float32),
            pltpu.VMEM((2, KK, BN), jnp.float32),
            pltpu.VMEM((2, KK, BN), e5m2),
            pltpu.SemaphoreType.DMA((N_DEV - 1,)),
            pltpu.SemaphoreType.DMA((N_DEV,)),
            pltpu.SemaphoreType.DMA((2,)),
            pltpu.SemaphoreType.DMA((2, KH)),
            pltpu.SemaphoreType.DMA((nb_half,)),
        ],
        compiler_params=pltpu.CompilerParams(
            dimension_semantics=("arbitrary", "arbitrary", "arbitrary"),
            vmem_limit_bytes=63 * 1024 * 1024,
        ),
    )(x, w_mat, scale_x, scale_w)
